# baseline (speedup 1.0000x reference)
import functools
import math

import jax
import jax.numpy as jnp
import numpy as np
from jax.experimental import pallas as pl
from jax.experimental.pallas import tpu as pltpu

_EMB = 64
_HEADS = 2


def _ceil_to(x, m):
    return (x + m - 1) // m * m


def _mlp_body(n_stages, acts, x_ref, *refs):
    o_ref = refs[-1]
    h = x_ref[...]
    for i in range(n_stages):
        W = refs[2 * i][...]
        b = refs[2 * i + 1][...]
        h = jnp.dot(h, W, preferred_element_type=jnp.float32) + b[None, :]
        a = acts[i]
        if a == "leaky":
            h = jnp.where(h > 0, h, 0.01 * h)
        elif a == "relu":
            h = jnp.maximum(h, 0.0)
    o_ref[...] = h


def _mlp_pallas(x, stages, acts, block_rows=2048):
    R, K = x.shape
    Rp = _ceil_to(R, block_rows)
    if Rp != R:
        x = jnp.pad(x, ((0, Rp - R), (0, 0)))
    n = len(stages)
    out_dim = stages[-1][0].shape[1]
    in_specs = [pl.BlockSpec((block_rows, K), lambda i: (i, 0))]
    args = [x]
    for (W, b) in stages:
        in_specs.append(pl.BlockSpec(W.shape, lambda i: (0, 0)))
        in_specs.append(pl.BlockSpec(b.shape, lambda i: (0,)))
        args.append(W)
        args.append(b)
    out = pl.pallas_call(
        functools.partial(_mlp_body, n, tuple(acts)),
        grid=(Rp // block_rows,),
        in_specs=in_specs,
        out_specs=pl.BlockSpec((block_rows, out_dim), lambda i: (i, 0)),
        out_shape=jax.ShapeDtypeStruct((Rp, out_dim), jnp.float32),
    )(*args)
    return out[:R]


def _linear(x, Wb, act="none", block_rows=2048):
    return _mlp_pallas(x, [Wb], [act], block_rows)


def _segment_softmax(logits, seg, num_segments):
    m = jax.ops.segment_max(logits, seg, num_segments=num_segments)
    m = jnp.where(jnp.isfinite(m), m, 0.0)
    e = jnp.exp(logits - m[seg])
    s = jax.ops.segment_sum(e, seg, num_segments=num_segments)
    return e / (s[seg] + 1e-16)


def _graph_layernorm(x, batch, num_graphs, eps=1e-5):
    F = x.shape[-1]
    cnt = jax.ops.segment_sum(jnp.ones(x.shape[0], x.dtype), batch, num_segments=num_graphs)
    norm = jnp.clip(cnt, 1.0) * F
    mean = jax.ops.segment_sum(x, batch, num_segments=num_graphs).sum(-1) / norm
    xc = x - mean[batch][:, None]
    var = jax.ops.segment_sum(xc * xc, batch, num_segments=num_graphs).sum(-1) / norm
    return xc / jnp.sqrt(var + eps)[batch][:, None]


def kernel(x, edge_attr, cond, params, edge_index, batch, non_edge_index):
    N = x.shape[0]
    B = cond.shape[0]
    Nt = N + B
    C = _EMB

    o = _mlp_pallas(x, params['x2h'], ["leaky", "leaky", "none"])
    e = _mlp_pallas(edge_attr, params['e2h'], ["leaky", "leaky", "none"])
    c = _mlp_pallas(cond, params['c2h'], ["leaky", "leaky", "none"], block_rows=64)

    u = jnp.arange(N, dtype=edge_index.dtype)
    v = batch.astype(edge_index.dtype) + N
    aug_ei = jnp.concatenate([edge_index, jnp.stack([u, v]), jnp.stack([v, u])], axis=1)
    e_p = jnp.zeros((2 * N, _EMB), x.dtype).at[:, 0].set(1.0)
    aug_e = jnp.concatenate([e, e_p], axis=0)
    dst0 = aug_ei[1]
    cnt = jax.ops.segment_sum(jnp.ones(dst0.shape[0], x.dtype), dst0, num_segments=Nt)
    loop_attr = jax.ops.segment_sum(aug_e, dst0, num_segments=Nt) / jnp.clip(cnt, 1.0)[:, None]
    loop_idx = jnp.arange(Nt, dtype=edge_index.dtype)
    aug_ei = jnp.concatenate([aug_ei, jnp.stack([loop_idx, loop_idx])], axis=1)
    aug_e = jnp.concatenate([aug_e, loop_attr], axis=0)
    aug_batch = jnp.concatenate([batch, jnp.arange(B, dtype=batch.dtype)], axis=0)
    o = jnp.concatenate([o, c], axis=0)
    src = aug_ei[0]
    dst = aug_ei[1]

    for lp in params['layers']:
        msg = jax.nn.relu(o[src] + aug_e) + 1e-7
        aggm = jax.ops.segment_sum(msg, dst, num_segments=Nt)
        agg = _linear(aggm + o, lp['gen'])
        h = jnp.concatenate([o, agg], axis=1)
        q = _linear(h, lp['lin_q']).reshape(Nt, _HEADS, C)
        k = _linear(h, lp['lin_k']).reshape(Nt, _HEADS, C)
        vv = _linear(h, lp['lin_v']).reshape(Nt, _HEADS, C)
        eh = _linear(aug_e, (lp['lin_edge'], jnp.zeros((2 * C,), jnp.float32))).reshape(-1, _HEADS, C)
        kj = k[src] + eh
        alpha = (q[dst] * kj).sum(-1) / np.sqrt(C).astype(np.float32)
        alpha = _segment_softmax(alpha, dst, Nt)
        outm = (vv[src] + eh) * alpha[..., None]
        out = jax.ops.segment_sum(outm, dst, num_segments=Nt).reshape(Nt, _HEADS * C)
        out = out + _linear(h, lp['lin_skip'])
        o = _graph_layernorm(o + _linear(out, lp['lin']), aug_batch, B)
        o = _graph_layernorm(o + _mlp_pallas(o, lp['ff'], ["leaky", "none"]), aug_batch, B)

    on = o[:N]
    oc = o[N:]
    cnt_n = jnp.clip(jax.ops.segment_sum(jnp.ones(N, x.dtype), batch, num_segments=B), 1.0)
    pooled = jax.ops.segment_sum(on, batch, num_segments=B) / cnt_n[:, None]
    glob = jnp.concatenate([pooled, oc, c], axis=1)
    o_final = jnp.concatenate([on, c[batch]], axis=1)
    hd = params['heads']
    ne_row, ne_col = non_edge_index[0], non_edge_index[1]
    e_row, e_col = edge_index[0, ::2], edge_index[1, ::2]
    stop = _linear(glob, hd['stop'], block_rows=64)
    add_node = _linear(o_final, hd['add_node'])
    set_node_attr = _linear(o_final, hd['set_node_attr'])
    add_edge = _linear(o_final[ne_row] + o_final[ne_col], hd['add_edge'])
    set_edge_attr = _linear(o_final[e_row] + o_final[e_col], hd['set_edge_attr'])
    reward = _linear(glob, hd['reward'], block_rows=64)
    return (stop, add_node, set_node_attr, add_edge, set_edge_attr, reward)


# speedup vs baseline: 1.0263x; 1.0263x over previous
import functools

import jax
import jax.numpy as jnp
import numpy as np
from jax.experimental import pallas as pl
from jax.experimental.pallas import tpu as pltpu

_EMB = 64
_HEADS = 2
_NEG = -1e30


def _ceil_to(x, m):
    return (x + m - 1) // m * m


def _mlp_body(n_stages, acts, x_ref, *refs):
    o_ref = refs[-1]
    h = x_ref[...]
    for i in range(n_stages):
        W = refs[2 * i][...]
        b = refs[2 * i + 1][...]
        h = jnp.dot(h, W, preferred_element_type=jnp.float32) + b[None, :]
        a = acts[i]
        if a == "leaky":
            h = jnp.where(h > 0, h, 0.01 * h)
        elif a == "relu":
            h = jnp.maximum(h, 0.0)
    o_ref[...] = h


def _mlp_pallas(x, stages, acts, block_rows=2048):
    R, K = x.shape
    Rp = _ceil_to(R, block_rows)
    if Rp != R:
        x = jnp.pad(x, ((0, Rp - R), (0, 0)))
    n = len(stages)
    out_dim = stages[-1][0].shape[1]
    in_specs = [pl.BlockSpec((block_rows, K), lambda i: (i, 0))]
    args = [x]
    for (W, b) in stages:
        in_specs.append(pl.BlockSpec(W.shape, lambda i: (0, 0)))
        in_specs.append(pl.BlockSpec(b.shape, lambda i: (0,)))
        args.append(W)
        args.append(b)
    out = pl.pallas_call(
        functools.partial(_mlp_body, n, tuple(acts)),
        grid=(Rp // block_rows,),
        in_specs=in_specs,
        out_specs=pl.BlockSpec((block_rows, out_dim), lambda i: (i, 0)),
        out_shape=jax.ShapeDtypeStruct((Rp, out_dim), jnp.float32),
    )(*args)
    return out[:R]


def _linear(x, Wb, act="none", block_rows=2048):
    return _mlp_pallas(x, [Wb], [act], block_rows)


def _otmm_body(at_ref, x_ref, o_ref):
    @pl.when(pl.program_id(0) == 0)
    def _():
        o_ref[...] = jnp.zeros_like(o_ref)
    o_ref[...] += jnp.dot(at_ref[...], x_ref[...],
                          preferred_element_type=jnp.float32)


def _otmm(at, x, block_rows=2048):
    return at @ x
    B, N = at.shape
    _, F = x.shape
    Np = _ceil_to(N, block_rows)
    if Np != N:
        at = jnp.pad(at, ((0, 0), (0, Np - N)))
        x = jnp.pad(x, ((0, Np - N), (0, 0)))
    return pl.pallas_call(
        _otmm_body,
        grid=(Np // block_rows,),
        in_specs=[pl.BlockSpec((B, block_rows), lambda i: (0, i)),
                  pl.BlockSpec((block_rows, F), lambda i: (i, 0))],
        out_specs=pl.BlockSpec((B, F), lambda i: (0, 0)),
        out_shape=jax.ShapeDtypeStruct((B, F), jnp.float32),
    )(at, x)


def _graph_ln(y, oh_t, batch, cnt_b, N, B, eps=1e-5):
    F = y.shape[-1]
    norm = (cnt_b + 1.0) * F
    batch_all = jnp.concatenate([batch, jnp.arange(B, dtype=batch.dtype)])
    sum_b = _otmm(oh_t, y[:N]) + y[N:]
    mean_b = sum_b.sum(-1) / norm
    yc = y - mean_b[batch_all][:, None]
    var_b = (_otmm(oh_t, yc[:N] * yc[:N]) + yc[N:] * yc[N:]).sum(-1) / norm
    inv = 1.0 / jnp.sqrt(var_b + eps)
    return yc * inv[batch_all][:, None]


def kernel(x, edge_attr, cond, params, edge_index, batch, non_edge_index):
    N = x.shape[0]
    B = cond.shape[0]
    Nt = N + B
    C = _EMB
    f32 = jnp.float32

    o = _mlp_pallas(x, params['x2h'], ["leaky", "leaky", "none"])
    e = _mlp_pallas(edge_attr, params['e2h'], ["leaky", "leaky", "none"])
    c = _mlp_pallas(cond, params['c2h'], ["leaky", "leaky", "none"], block_rows=64)

    oh = (batch[:, None] == jnp.arange(B, dtype=batch.dtype)[None, :]).astype(f32)
    oh_t = oh.T
    cnt_b = oh.sum(0)
    src_r = edge_index[0]
    dst_r = edge_index[1]
    indeg_r = jax.ops.segment_sum(jnp.ones((src_r.shape[0],), f32), dst_r, num_segments=N)
    e_p1 = jnp.zeros((_EMB,), f32).at[0].set(1.0)
    loop_sum_r = jax.ops.segment_sum(e, dst_r, num_segments=N)
    cnt0_n = indeg_r + 1.0
    loop_attr_n = (loop_sum_r + e_p1[None, :]) / cnt0_n[:, None]
    loop_attr_c = (cnt_b[:, None] / jnp.clip(cnt_b, 1.0)[:, None]) * e_p1[None, :]
    loop_attr = jnp.concatenate([loop_attr_n, loop_attr_c], axis=0)

    u = jnp.arange(N, dtype=edge_index.dtype)
    v = batch.astype(edge_index.dtype) + N
    aug_ei = jnp.concatenate([edge_index, jnp.stack([u, v]), jnp.stack([v, u])], axis=1)
    e_p = jnp.zeros((2 * N, _EMB), x.dtype).at[:, 0].set(1.0)
    aug_e = jnp.concatenate([e, e_p], axis=0)
    loop_idx = jnp.arange(Nt, dtype=edge_index.dtype)
    aug_ei = jnp.concatenate([aug_ei, jnp.stack([loop_idx, loop_idx])], axis=1)
    aug_e = jnp.concatenate([aug_e, loop_attr], axis=0)
    aug_batch = jnp.concatenate([batch, jnp.arange(B, dtype=batch.dtype)], axis=0)
    o = jnp.concatenate([o, c], axis=0)
    src = aug_ei[0]
    dst = aug_ei[1]

    for lp in params['layers']:
        o_nodes, o_c = o[:N], o[N:]
        msg_r = jax.nn.relu(o[src_r] + e) + 1e-7
        aggm_r = jax.ops.segment_sum(msg_r, dst_r, num_segments=N)
        o_c_rows = _linear(oh, (o_c, jnp.zeros((C,), f32)))
        aggm_vu = jax.nn.relu(o_c_rows + e_p1[None, :]) + 1e-7
        aggm_l = jax.nn.relu(o + loop_attr) + 1e-7
        aggm_uv = _otmm(oh_t, jax.nn.relu(o_nodes + e_p1[None, :])) + cnt_b[:, None] * 1e-7
        aggm = jnp.concatenate([aggm_r + aggm_vu, aggm_uv], axis=0) + aggm_l
        agg = _linear(aggm + o, lp['gen'])
        h = jnp.concatenate([o, agg], axis=1)
        q = _linear(h, lp['lin_q']).reshape(Nt, _HEADS, C)
        k = _linear(h, lp['lin_k']).reshape(Nt, _HEADS, C)
        vv = _linear(h, lp['lin_v']).reshape(Nt, _HEADS, C)
        eh = _linear(aug_e, (lp['lin_edge'], jnp.zeros((2 * C,), jnp.float32))).reshape(-1, _HEADS, C)
        kj = k[src] + eh
        alpha = (q[dst] * kj).sum(-1) / np.sqrt(C).astype(np.float32)
        m = jax.ops.segment_max(alpha, dst, num_segments=Nt)
        m = jnp.where(jnp.isfinite(m), m, 0.0)
        ew = jnp.exp(alpha - m[dst])
        sden = jax.ops.segment_sum(ew, dst, num_segments=Nt)
        alpha = ew / (sden[dst] + 1e-16)
        outm = (vv[src] + eh) * alpha[..., None]
        out = jax.ops.segment_sum(outm, dst, num_segments=Nt).reshape(Nt, _HEADS * C)
        out = out + _linear(h, lp['lin_skip'])
        o = _graph_ln(o + _linear(out, lp['lin']), oh_t, batch, cnt_b, N, B)
        o = _graph_ln(o + _mlp_pallas(o, lp['ff'], ["leaky", "none"]), oh_t, batch, cnt_b, N, B)

    on = o[:N]
    oc = o[N:]
    cnt_n = jnp.clip(cnt_b, 1.0)
    pooled = _otmm(oh_t, on) / cnt_n[:, None]
    glob = jnp.concatenate([pooled, oc, c], axis=1)
    c_rows = _linear(oh, (c, jnp.zeros((C,), f32)))
    o_final = jnp.concatenate([on, c_rows], axis=1)
    hd = params['heads']
    ne_row, ne_col = non_edge_index[0], non_edge_index[1]
    e_row, e_col = edge_index[0, ::2], edge_index[1, ::2]
    stop = _linear(glob, hd['stop'], block_rows=64)
    add_node = _linear(o_final, hd['add_node'])
    set_node_attr = _linear(o_final, hd['set_node_attr'])
    add_edge = _linear(o_final[ne_row] + o_final[ne_col], hd['add_edge'])
    set_edge_attr = _linear(o_final[e_row] + o_final[e_col], hd['set_edge_attr'])
    reward = _linear(glob, hd['reward'], block_rows=64)
    return (stop, add_node, set_node_attr, add_edge, set_edge_attr, reward)


# speedup vs baseline: 4.3967x; 4.2838x over previous
import functools

import jax
import jax.numpy as jnp
import numpy as np
from jax.experimental import pallas as pl
from jax.experimental.pallas import tpu as pltpu

_EMB = 64
_HEADS = 2
_NEG = -1e30


def _ceil_to(x, m):
    return (x + m - 1) // m * m


def _mlp_body(n_stages, acts, x_ref, *refs):
    o_ref = refs[-1]
    h = x_ref[...]
    for i in range(n_stages):
        W = refs[2 * i][...]
        b = refs[2 * i + 1][...]
        h = jnp.dot(h, W, preferred_element_type=jnp.float32) + b[None, :]
        a = acts[i]
        if a == "leaky":
            h = jnp.where(h > 0, h, 0.01 * h)
        elif a == "relu":
            h = jnp.maximum(h, 0.0)
    o_ref[...] = h


def _mlp_pallas(x, stages, acts, block_rows=2048):
    R, K = x.shape
    Rp = _ceil_to(R, block_rows)
    if Rp != R:
        x = jnp.pad(x, ((0, Rp - R), (0, 0)))
    n = len(stages)
    out_dim = stages[-1][0].shape[1]
    in_specs = [pl.BlockSpec((block_rows, K), lambda i: (i, 0))]
    args = [x]
    for (W, b) in stages:
        in_specs.append(pl.BlockSpec(W.shape, lambda i: (0, 0)))
        in_specs.append(pl.BlockSpec(b.shape, lambda i: (0,)))
        args.append(W)
        args.append(b)
    out = pl.pallas_call(
        functools.partial(_mlp_body, n, tuple(acts)),
        grid=(Rp // block_rows,),
        in_specs=in_specs,
        out_specs=pl.BlockSpec((block_rows, out_dim), lambda i: (i, 0)),
        out_shape=jax.ShapeDtypeStruct((Rp, out_dim), jnp.float32),
    )(*args)
    return out[:R]


def _linear(x, Wb, act="none", block_rows=2048):
    return _mlp_pallas(x, [Wb], [act], block_rows)


def _otmm_body(at_ref, x_ref, o_ref):
    @pl.when(pl.program_id(0) == 0)
    def _():
        o_ref[...] = jnp.zeros_like(o_ref)
    o_ref[...] += jnp.dot(at_ref[...], x_ref[...],
                          preferred_element_type=jnp.float32)


def _otmm(at, x, block_rows=2048):
    return at @ x
    B, N = at.shape
    _, F = x.shape
    Np = _ceil_to(N, block_rows)
    if Np != N:
        at = jnp.pad(at, ((0, 0), (0, Np - N)))
        x = jnp.pad(x, ((0, Np - N), (0, 0)))
    return pl.pallas_call(
        _otmm_body,
        grid=(Np // block_rows,),
        in_specs=[pl.BlockSpec((B, block_rows), lambda i: (0, i)),
                  pl.BlockSpec((block_rows, F), lambda i: (i, 0))],
        out_specs=pl.BlockSpec((B, F), lambda i: (0, 0)),
        out_shape=jax.ShapeDtypeStruct((B, F), jnp.float32),
    )(at, x)


def _graph_ln(y, oh_t, batch, cnt_b, N, B, eps=1e-5):
    F = y.shape[-1]
    norm = (cnt_b + 1.0) * F
    batch_all = jnp.concatenate([batch, jnp.arange(B, dtype=batch.dtype)])
    sum_b = _otmm(oh_t, y[:N]) + y[N:]
    mean_b = sum_b.sum(-1) / norm
    yc = y - mean_b[batch_all][:, None]
    var_b = (_otmm(oh_t, yc[:N] * yc[:N]) + yc[N:] * yc[N:]).sum(-1) / norm
    inv = 1.0 / jnp.sqrt(var_b + eps)
    return yc * inv[batch_all][:, None]


from jax import lax
from jax.experimental.pallas import tpu_sc as plsc

_N = 10000
_E = 320000
_K = 80
_EPT = _E // 32
_NCH = _EPT // _K
_RPT = 632


def _init_acc(z_hbm, acc, sid):
    r0 = sid * _RPT
    pltpu.sync_copy(z_hbm.at[pl.ds(0, 520)], acc.at[pl.ds(r0, 520)])
    @pl.when(sid < 15)
    def _():
        pltpu.sync_copy(z_hbm.at[pl.ds(520, 112)], acc.at[pl.ds(r0 + 520, 112)])


def _flush_acc(acc, out_hbm, cid, sid):
    r0 = sid * _RPT
    pltpu.sync_copy(acc.at[pl.ds(r0, 520)], out_hbm.at[cid, pl.ds(r0, 520)])
    @pl.when(sid < 15)
    def _():
        pltpu.sync_copy(acc.at[pl.ds(r0 + 520, 112)],
                        out_hbm.at[cid, pl.ds(r0 + 520, 112)])

_SC_CACHE = {}


def _sc_mesh():
    return plsc.VectorSubcoreMesh(core_axis_name="c", subcore_axis_name="s")


def _wid():
    return lax.axis_index("s") * 2 + lax.axis_index("c")


def _agg_body(o_hbm, e_hbm, src_hbm, dst_hbm, z_hbm, out_hbm,
              sidx, didx, obuf, ebuf, acc, sem):
    cid = lax.axis_index("c")
    sid = lax.axis_index("s")
    w = _wid()
    _init_acc(z_hbm, acc, sid)
    plsc.subcore_barrier()

    def chunk(j, carry):
        base = w * _EPT + j * _K
        pltpu.sync_copy(src_hbm.at[pl.ds(base, _K)], sidx)
        pltpu.sync_copy(dst_hbm.at[pl.ds(base, _K)], didx)
        pltpu.sync_copy(e_hbm.at[pl.ds(base, _K)], ebuf)
        pltpu.async_copy(o_hbm.at[sidx], obuf, sem).wait()

        def row(r, carry2):
            for v in range(4):
                sl = pl.ds(v * 16, 16)
                m = jnp.maximum(obuf[r, sl] + ebuf[r, sl], 0.0) + 1e-7
                obuf[r, sl] = m
            return carry2
        lax.fori_loop(0, _K, row, 0, unroll=4)
        pltpu.sync_copy(obuf, acc.at[didx], add=True)
        return carry
    lax.fori_loop(0, _NCH, chunk, 0)
    plsc.subcore_barrier()
    _flush_acc(acc, out_hbm, cid, sid)


def agg_kernel(*args):
    if 'agg' not in _SC_CACHE:
        _SC_CACHE['agg'] = pl.kernel(
            _agg_body, mesh=_sc_mesh(),
            out_type=jax.ShapeDtypeStruct((2, _N, 128), jnp.float32),
            scratch_types=[
                pltpu.VMEM((_K,), jnp.int32),
                pltpu.VMEM((_K,), jnp.int32),
                pltpu.VMEM((_K, 128), jnp.float32),
                pltpu.VMEM((_K, 64), jnp.float32),
                pltpu.VMEM_SHARED((_N, 128), jnp.float32),
                pltpu.SemaphoreType.DMA,
            ])
    return _SC_CACHE['agg'](*args)


def _iota16():
    return lax.iota(jnp.int32, 16)


_GDN = lax.GatherDimensionNumbers(
    offset_dims=(), collapsed_slice_dims=(0,), start_index_map=(0,))


def _gather16(v, idx):
    return lax.gather(v, idx[:, None], _GDN, (1,),
                      mode=lax.GatherScatterMode.PROMISE_IN_BOUNDS)


def _lane_sum(v):
    idx = _iota16()
    for sh in (8, 4, 2, 1):
        v = v + _gather16(v, jnp.bitwise_xor(idx, sh))
    return v


def _att1_body(q_hbm, k_hbm, eh_hbm, src_hbm, dst_hbm, al0_hbm, al1_hbm,
               sidx, didx, qbuf, kbuf, ehbuf, al0buf, al1buf, sem):
    w = _wid()
    scale = jnp.float32(0.125)

    def chunk(j, carry):
        base = w * _EPT + j * _K
        pltpu.sync_copy(src_hbm.at[pl.ds(base, _K)], sidx)
        pltpu.sync_copy(dst_hbm.at[pl.ds(base, _K)], didx)
        pltpu.sync_copy(eh_hbm.at[pl.ds(base, _K)], ehbuf)
        pltpu.async_copy(k_hbm.at[sidx], kbuf, sem).wait()
        pltpu.async_copy(q_hbm.at[didx], qbuf, sem).wait()

        def grp(g, carry2):
            a0 = jnp.zeros((16,), jnp.float32)
            a1 = jnp.zeros((16,), jnp.float32)
            for jj in range(16):
                r = g * 16 + jj
                acc0 = jnp.zeros((16,), jnp.float32)
                acc1 = jnp.zeros((16,), jnp.float32)
                for v in range(4):
                    sl = pl.ds(v * 16, 16)
                    sl1 = pl.ds(64 + v * 16, 16)
                    acc0 = acc0 + qbuf[r, sl] * (kbuf[r, sl] + ehbuf[r, sl])
                    acc1 = acc1 + qbuf[r, sl1] * (kbuf[r, sl1] + ehbuf[r, sl1])
                lane = _iota16() == jj
                a0 = jnp.where(lane, _lane_sum(acc0) * scale, a0)
                a1 = jnp.where(lane, _lane_sum(acc1) * scale, a1)
            al0buf[pl.ds(g * 16, 16)] = a0
            al1buf[pl.ds(g * 16, 16)] = a1
            return carry2
        lax.fori_loop(0, _K // 16, grp, 0)
        pltpu.sync_copy(al0buf, al0_hbm.at[pl.ds(base, _K)])
        pltpu.sync_copy(al1buf, al1_hbm.at[pl.ds(base, _K)])
        return carry
    lax.fori_loop(0, _NCH, chunk, 0)


def att1_kernel(*args):
    if 'att1' not in _SC_CACHE:
        _SC_CACHE['att1'] = pl.kernel(
            _att1_body, mesh=_sc_mesh(),
            out_type=(jax.ShapeDtypeStruct((_E,), jnp.float32),
                      jax.ShapeDtypeStruct((_E,), jnp.float32)),
            scratch_types=[
                pltpu.VMEM((_K,), jnp.int32),
                pltpu.VMEM((_K,), jnp.int32),
                pltpu.VMEM((_K, 128), jnp.float32),
                pltpu.VMEM((_K, 128), jnp.float32),
                pltpu.VMEM((_K, 128), jnp.float32),
                pltpu.VMEM((_K,), jnp.float32),
                pltpu.VMEM((_K,), jnp.float32),
                pltpu.SemaphoreType.DMA,
            ])
    return _SC_CACHE['att1'](*args)


def _att2a_body(vv_hbm, eh_hbm, alm0_hbm, alm1_hbm, src_hbm, dst_hbm, z_hbm,
                out_hbm,
                sidx, didx, vvbuf, ehbuf, al0buf, al1buf, msgbuf, acc, sem):
    cid = lax.axis_index("c")
    sid = lax.axis_index("s")
    w = _wid()
    _init_acc(z_hbm, acc, sid)
    plsc.subcore_barrier()

    def chunk(j, carry):
        base = w * _EPT + j * _K
        pltpu.sync_copy(src_hbm.at[pl.ds(base, _K)], sidx)
        pltpu.sync_copy(dst_hbm.at[pl.ds(base, _K)], didx)
        pltpu.sync_copy(alm0_hbm.at[pl.ds(base, _K)], al0buf)
        pltpu.sync_copy(alm1_hbm.at[pl.ds(base, _K)], al1buf)
        pltpu.sync_copy(eh_hbm.at[pl.ds(base, _K)], ehbuf)
        pltpu.async_copy(vv_hbm.at[sidx], vvbuf, sem).wait()

        def grp(g, carry2):
            w0 = jnp.exp(al0buf[pl.ds(g * 16, 16)])
            w1 = jnp.exp(al1buf[pl.ds(g * 16, 16)])
            for jj in range(16):
                r = g * 16 + jj
                b0 = jnp.full((16,), w0[jj], jnp.float32)
                b1 = jnp.full((16,), w1[jj], jnp.float32)
                for v in range(4):
                    sl = pl.ds(v * 16, 16)
                    sl1 = pl.ds(64 + v * 16, 16)
                    msgbuf[r, sl] = (vvbuf[r, sl] + ehbuf[r, sl]) * b0
                    msgbuf[r, sl1] = (vvbuf[r, sl1] + ehbuf[r, sl1]) * b1
            return carry2
        lax.fori_loop(0, _K // 16, grp, 0)
        pltpu.sync_copy(msgbuf, acc.at[didx], add=True)
        return carry
    lax.fori_loop(0, _NCH, chunk, 0)
    plsc.subcore_barrier()
    _flush_acc(acc, out_hbm, cid, sid)


def att2a_kernel(*args):
    if 'att2a' not in _SC_CACHE:
        _SC_CACHE['att2a'] = pl.kernel(
            _att2a_body, mesh=_sc_mesh(),
            out_type=jax.ShapeDtypeStruct((2, _N, 128), jnp.float32),
            scratch_types=[
                pltpu.VMEM((_K,), jnp.int32),
                pltpu.VMEM((_K,), jnp.int32),
                pltpu.VMEM((_K, 128), jnp.float32),
                pltpu.VMEM((_K, 128), jnp.float32),
                pltpu.VMEM((_K,), jnp.float32),
                pltpu.VMEM((_K,), jnp.float32),
                pltpu.VMEM((_K, 128), jnp.float32),
                pltpu.VMEM_SHARED((_N, 128), jnp.float32),
                pltpu.SemaphoreType.DMA,
            ])
    return _SC_CACHE['att2a'](*args)


def _atts_body(alm0_hbm, alm1_hbm, dst_hbm, z_hbm, out_hbm,
               didx, al0buf, al1buf, msgbuf, acc, sem):
    cid = lax.axis_index("c")
    sid = lax.axis_index("s")
    w = _wid()
    _init_acc(z_hbm, acc, sid)

    def zrow(r, carry):
        for v in range(8):
            msgbuf[r, pl.ds(v * 16, 16)] = jnp.zeros((16,), jnp.float32)
        return carry
    lax.fori_loop(0, _K, zrow, 0, unroll=8)
    plsc.subcore_barrier()
    lane0 = _iota16() == 0
    lane1 = _iota16() == 1

    def chunk(j, carry):
        base = w * _EPT + j * _K
        pltpu.sync_copy(dst_hbm.at[pl.ds(base, _K)], didx)
        pltpu.sync_copy(alm0_hbm.at[pl.ds(base, _K)], al0buf)
        pltpu.sync_copy(alm1_hbm.at[pl.ds(base, _K)], al1buf)

        def grp(g, carry2):
            w0 = jnp.exp(al0buf[pl.ds(g * 16, 16)])
            w1 = jnp.exp(al1buf[pl.ds(g * 16, 16)])
            for jj in range(16):
                r = g * 16 + jj
                b = jnp.where(lane0, jnp.full((16,), w0[jj], jnp.float32),
                              jnp.where(lane1, jnp.full((16,), w1[jj], jnp.float32),
                                        jnp.zeros((16,), jnp.float32)))
                msgbuf[r, pl.ds(0, 16)] = b
            return carry2
        lax.fori_loop(0, _K // 16, grp, 0)
        pltpu.sync_copy(msgbuf, acc.at[didx], add=True)
        return carry
    lax.fori_loop(0, _NCH, chunk, 0)
    plsc.subcore_barrier()
    _flush_acc(acc, out_hbm, cid, sid)


def atts_kernel(*args):
    if 'atts' not in _SC_CACHE:
        _SC_CACHE['atts'] = pl.kernel(
            _atts_body, mesh=_sc_mesh(),
            out_type=jax.ShapeDtypeStruct((2, _N, 128), jnp.float32),
            scratch_types=[
                pltpu.VMEM((_K,), jnp.int32),
                pltpu.VMEM((_K,), jnp.float32),
                pltpu.VMEM((_K,), jnp.float32),
                pltpu.VMEM((_K, 128), jnp.float32),
                pltpu.VMEM_SHARED((_N, 128), jnp.float32),
                pltpu.SemaphoreType.DMA,
            ])
    return _SC_CACHE['atts'](*args)


def _sum_body(epk_hbm, dst_hbm, z_hbm, out_hbm, didx, msgbuf, acc, sem):
    cid = lax.axis_index("c")
    sid = lax.axis_index("s")
    w = _wid()
    _init_acc(z_hbm, acc, sid)
    plsc.subcore_barrier()

    def chunk(j, carry):
        base = w * _EPT + j * _K
        pltpu.sync_copy(dst_hbm.at[pl.ds(base, _K)], didx)
        pltpu.sync_copy(epk_hbm.at[pl.ds(base, _K)], msgbuf)
        pltpu.sync_copy(msgbuf, acc.at[didx], add=True)
        return carry
    lax.fori_loop(0, _NCH, chunk, 0)
    plsc.subcore_barrier()
    _flush_acc(acc, out_hbm, cid, sid)


def sum_kernel(*args):
    if 'sum' not in _SC_CACHE:
        _SC_CACHE['sum'] = pl.kernel(
            _sum_body, mesh=_sc_mesh(),
            out_type=jax.ShapeDtypeStruct((2, _N, 128), jnp.float32),
            scratch_types=[
                pltpu.VMEM((_K,), jnp.int32),
                pltpu.VMEM((_K, 128), jnp.float32),
                pltpu.VMEM_SHARED((_N, 128), jnp.float32),
                pltpu.SemaphoreType.DMA,
            ])
    return _SC_CACHE['sum'](*args)


def kernel(x, edge_attr, cond, params, edge_index, batch, non_edge_index):
    N = x.shape[0]
    B = cond.shape[0]
    Nt = N + B
    C = _EMB
    H = _HEADS
    scale = np.float32(1.0 / np.sqrt(C))
    f32 = jnp.float32

    o_n = _mlp_pallas(x, params['x2h'], ["leaky", "leaky", "none"])
    e = _mlp_pallas(edge_attr, params['e2h'], ["leaky", "leaky", "none"])
    c = _mlp_pallas(cond, params['c2h'], ["leaky", "leaky", "none"], block_rows=64)

    oh = (batch[:, None] == jnp.arange(B, dtype=batch.dtype)[None, :]).astype(f32)
    oh_t = oh.T
    cnt_b = oh.sum(0)

    src_r = edge_index[0]
    dst_r = edge_index[1]
    z128 = jnp.zeros((_RPT, 128), f32)
    E = src_r.shape[0]
    epk = jnp.concatenate([e, jnp.ones((E, 1), f32), jnp.zeros((E, 63), f32)], axis=1)
    es2 = sum_kernel(epk, dst_r, z128)
    es = es2[0] + es2[1]
    loop_sum_r = es[:, :64]
    indeg_r = es[:, 64]

    e_p = jnp.zeros((_EMB,), f32).at[0].set(1.0)
    cnt0_n = indeg_r + 1.0
    loop_attr_n = (loop_sum_r + e_p[None, :]) / cnt0_n[:, None]
    loop_attr_c = (cnt_b[:, None] / jnp.clip(cnt_b, 1.0)[:, None]) * e_p[None, :]
    loop_attr = jnp.concatenate([loop_attr_n, loop_attr_c], axis=0)

    o = jnp.concatenate([o_n, c], axis=0)

    for lp in params['layers']:
        o_nodes, o_c = o[:N], o[N:]
        o_dep, _ = jax.lax.optimization_barrier((o, loop_attr))
        o128 = jnp.pad(o_dep, ((0, 0), (0, 64)))
        ag2 = agg_kernel(o128, e, src_r, dst_r, z128)
        aggm_r = (ag2[0] + ag2[1])[:, :64]
        o_c_rows = _linear(oh, (o_c, jnp.zeros((C,), f32)))
        aggm_vu = jax.nn.relu(o_c_rows + e_p[None, :]) + 1e-7
        aggm_l = jax.nn.relu(o + loop_attr) + 1e-7
        aggm_uv = _otmm(oh_t, jax.nn.relu(o_nodes + e_p[None, :])) + cnt_b[:, None] * 1e-7
        aggm = jnp.concatenate([aggm_r + aggm_vu, aggm_uv], axis=0) + aggm_l

        agg = _linear(aggm + o, lp['gen'])
        h = jnp.concatenate([o, agg], axis=1)
        q = _linear(h, lp['lin_q'])
        k = _linear(h, lp['lin_k'])
        vv = _linear(h, lp['lin_v'])
        W_E = lp['lin_edge']
        eh_p = W_E[0]
        eh_loop = _linear(loop_attr, (W_E, jnp.zeros((H * C,), f32)))
        eh_r = _linear(e, (W_E, jnp.zeros((H * C,), f32)))

        qh = q.reshape(Nt, H, C)
        kh = k.reshape(Nt, H, C)
        vh = vv.reshape(Nt, H, C)
        ehp_h = eh_p.reshape(H, C)
        ehl_h = eh_loop.reshape(Nt, H, C)

        al0, al1 = att1_kernel(q, k, eh_r, src_r, dst_r)
        al_r = jnp.stack([al0, al1], axis=1)
        k_c_rows = _linear(oh, (k[N:], jnp.zeros((H * C,), f32))).reshape(N, H, C)
        al_vu = ((qh[:N] * (k_c_rows + ehp_h[None])).sum(-1)) * scale
        al_l = ((qh * (kh + ehl_h)).sum(-1)) * scale
        q_c_rows = _linear(oh, (q[N:], jnp.zeros((H * C,), f32))).reshape(N, H, C)
        al_uv = ((q_c_rows * (kh[:N] + ehp_h[None])).sum(-1)) * scale
        al_uv, _ = jax.lax.optimization_barrier((al_uv, al0))

        G0 = jnp.max(al0)
        G1 = jnp.max(al1)
        s1acc = atts_kernel((al0 - G0) * 0.25, (al1 - G1) * 0.25, dst_r, z128)
        s1 = (s1acc[0] + s1acc[1])[:, :2]
        m_r = jnp.stack([G0 + 4.0 * jnp.log(s1[:, 0]),
                         G1 + 4.0 * jnp.log(s1[:, 1])], axis=1)
        m_n = jnp.maximum(jnp.maximum(m_r, al_vu), al_l[:N])
        m_uv = jnp.where(oh[:, :, None] > 0, al_uv[:, None, :], _NEG).max(0)
        m_c = jnp.maximum(m_uv, al_l[N:])
        m = jnp.concatenate([m_n, m_c], axis=0)

        alm0 = jnp.minimum(al0 - m[dst_r, 0], 0.0)
        alm1 = jnp.minimum(al1 - m[dst_r, 1], 0.0)
        outv2 = att2a_kernel(vv, eh_r, alm0, alm1, src_r, dst_r, z128)
        alms0, _ = jax.lax.optimization_barrier((alm0, outv2))
        sacc = atts_kernel(alms0, alm1, dst_r, z128)
        s_r = (sacc[0] + sacc[1])[:, :2]
        w_vu = jnp.exp(al_vu - m[:N])
        w_l = jnp.exp(al_l - m)
        w_uv = jnp.exp(al_uv - m[N:][batch])
        s_n = s_r + w_vu + w_l[:N]
        s_c = _otmm(oh_t, w_uv) + w_l[N:]
        s = jnp.concatenate([s_n, s_c], axis=0)

        out_n = (outv2[0] + outv2[1]).reshape(N, H, C)
        v_c_rows = _linear(oh, (vv[N:], jnp.zeros((H * C,), f32))).reshape(N, H, C)
        out_n = out_n + w_vu[:, :, None] * (v_c_rows + ehp_h[None])
        out_sl = w_l[:, :, None] * (vh + ehl_h)
        wuv_flat = (w_uv[:, :, None] * (vh[:N] + ehp_h[None])).reshape(N, H * C)
        out_c = _otmm(oh_t, wuv_flat).reshape(B, H, C)
        out = jnp.concatenate([out_n, out_c], axis=0) + out_sl
        out = (out / (s[:, :, None] + 1e-16)).reshape(Nt, H * C)

        out = out + _linear(h, lp['lin_skip'])
        o = _graph_ln(o + _linear(out, lp['lin']), oh_t, batch, cnt_b, N, B)
        o = _graph_ln(o + _mlp_pallas(o, lp['ff'], ["leaky", "none"]), oh_t, batch, cnt_b, N, B)

    on = o[:N]
    oc = o[N:]
    cnt_n = jnp.clip(cnt_b, 1.0)
    pooled = _otmm(oh_t, on) / cnt_n[:, None]
    glob = jnp.concatenate([pooled, oc, c], axis=1)
    c_rows = _linear(oh, (c, jnp.zeros((C,), f32)))
    o_final = jnp.concatenate([on, c_rows], axis=1)
    hd = params['heads']
    ne_row, ne_col = non_edge_index[0], non_edge_index[1]
    e_row, e_col = edge_index[0, ::2], edge_index[1, ::2]
    stop = _linear(glob, hd['stop'], block_rows=64)
    add_node = _linear(o_final, hd['add_node'])
    set_node_attr = _linear(o_final, hd['set_node_attr'])
    add_edge = _linear(o_final[ne_row] + o_final[ne_col], hd['add_edge'])
    set_edge_attr = _linear(o_final[e_row] + o_final[e_col], hd['set_edge_attr'])
    reward = _linear(glob, hd['reward'], block_rows=64)
    return (stop, add_node, set_node_attr, add_edge, set_edge_attr, reward)


def _otmm_max(al_uv, oh):
    masked = jnp.where(oh[:, :, None] > 0, al_uv[:, None, :], _NEG)
    return masked.max(0)
